# baseline (speedup 1.0000x reference)
import functools

import jax
import jax.numpy as jnp
from jax import lax
from jax.experimental import pallas as pl
from jax.experimental.pallas import tpu as pltpu
from jax.experimental.pallas import tpu_sc as plsc

SLICE = 64
L = 16
G = 5
IB = 128

SC_ROWS = 98304
TC_BLK = 2048


def _sc_slice_gather(n_sc, d, rows_per_w, chunk_rows, nc):
    n_chunks = rows_per_w // chunk_rows
    assert n_chunks % 2 == 0
    groups = chunk_rows // L
    n_batch = chunk_rows * G // IB
    gpr = d // L

    mesh = plsc.VectorSubcoreMesh(core_axis_name="c", subcore_axis_name="s")

    @functools.partial(
        pl.kernel,
        mesh=mesh,
        compiler_params=pltpu.CompilerParams(
            needs_layout_passes=False, use_tc_tiling_on_sc=False),
        out_type=jax.ShapeDtypeStruct((n_sc * SLICE,), jnp.float32),
        scratch_types=[
            pltpu.VMEM((chunk_rows * G, L), jnp.float32),
            pltpu.VMEM((chunk_rows * G, L), jnp.float32),
            pltpu.VMEM((chunk_rows * SLICE,), jnp.float32),
            pltpu.VMEM((chunk_rows * SLICE,), jnp.float32),
            pltpu.VMEM((chunk_rows * G,), jnp.int32),
            pltpu.VMEM((chunk_rows * G,), jnp.int32),
            pltpu.VMEM((chunk_rows,), jnp.int32),
            pltpu.VMEM((chunk_rows,), jnp.int32),
            pltpu.VMEM((chunk_rows,), jnp.int32),
            pltpu.VMEM((chunk_rows,), jnp.int32),
            pltpu.SemaphoreType.DMA,
            pltpu.SemaphoreType.DMA,
            pltpu.SemaphoreType.DMA,
            pltpu.SemaphoreType.DMA,
            pltpu.SemaphoreType.DMA,
            pltpu.SemaphoreType.DMA,
        ],
    )
    def k(tab_hbm, idx_hbm, out_hbm, gat0, gat1, out_v0, out_v1,
          ig0, ig1, idx_v0, idx_v1, u_v0, u_v1,
          sem_i0, sem_i1, sem_g0, sem_g1, sem_o0, sem_o1):
        gat = (gat0, gat1)
        out_v = (out_v0, out_v1)
        ig = (ig0, ig1)
        idx_v = (idx_v0, idx_v1)
        u_v = (u_v0, u_v1)
        sem_i = (sem_i0, sem_i1)
        sem_g = (sem_g0, sem_g1)
        sem_o = (sem_o0, sem_o1)
        wid = lax.axis_index("s") * nc + lax.axis_index("c")
        base_row = wid * rows_per_w
        iota = lax.iota(jnp.int32, L)

        def idx_copy(c, b):
            row0 = base_row + c * chunk_rows
            return pltpu.make_async_copy(
                idx_hbm.at[pl.ds(row0, chunk_rows)], idx_v[b], sem_i[b])

        def out_copy(c, b):
            row0 = base_row + c * chunk_rows
            return pltpu.make_async_copy(
                out_v[b],
                out_hbm.at[pl.ds(row0 * SLICE, chunk_rows * SLICE)],
                sem_o[b])

        def gather_copies(b):
            return [pltpu.make_async_copy(
                        tab_hbm.at[ig[b].at[pl.ds(kk * IB, IB)]],
                        gat[b].at[pl.ds(kk * IB, IB)],
                        sem_g[b])
                    for kk in range(n_batch)]

        def build(c, b):
            row0 = base_row + c * chunk_rows

            @plsc.parallel_loop(0, groups, 1)
            def _(g):
                svec = idx_v[b][pl.ds(g * L, L)]
                base = jnp.full((L,), (row0 + g * L) * gpr, jnp.int32) \
                    + iota * gpr + lax.shift_right_logical(svec, 4)
                pos0 = iota * G + (g * (L * G))
                for kk in range(G):
                    plsc.store_scatter(ig[b], [pos0 + kk], base + kk)
                u_v[b][pl.ds(g * L, L)] = lax.bitwise_and(svec, 15)

        def compute(b):
            @plsc.parallel_loop(0, groups, 1)
            def _(g):
                uvec = u_v[b][pl.ds(g * L, L)]
                for r in range(L):
                    ubc = jnp.take_along_axis(
                        uvec, jnp.full((L,), r, jnp.int32), axis=0)
                    q0 = iota - ubc
                    obase = jnp.full((L,), (g * L + r) * SLICE, jnp.int32)
                    for kk in range(G):
                        vals = gat[b][g * (L * G) + r * G + kk]
                        q = q0 + (kk * L)
                        m = (q >= 0) & (q < SLICE)
                        plsc.store_scatter(out_v[b], [obase + q], vals,
                                           mask=m)

        for b in range(2):
            idx_copy(b, b).start()
        for b in range(2):
            idx_copy(b, b).wait()
            build(b, b)
            for cp in gather_copies(b):
                cp.start()
            idx_copy(b + 2, b).start()

        def pair_body(i, carry):
            for b in range(2):
                c = i * 2 + b
                for cp in gather_copies(b):
                    cp.wait()

                @pl.when(i > 0)
                def _():
                    out_copy(c, b).wait()

                compute(b)
                out_copy(c, b).start()

                @pl.when(c + 2 < n_chunks)
                def _():
                    idx_copy(c + 2, b).wait()
                    build(c + 2, b)
                    for cp in gather_copies(b):
                        cp.start()

                    @pl.when(c + 4 < n_chunks)
                    def _():
                        idx_copy(c + 4, b).start()
            return carry

        lax.fori_loop(0, n_chunks // 2, pair_body, 0)
        for b in range(2):
            out_copy(n_chunks - 2 + b, b).wait()

    return k


def _tc_slice(n, d, row0, n_rows, blk):
    grid = (n_rows // blk,)
    off = row0 // blk

    def body(in_ref, idx_ref, out_ref):
        x = in_ref[...]
        s = idx_ref[...]
        j = lax.broadcasted_iota(jnp.int32, (blk, d), 1)
        col = jnp.minimum(s + j, d - 1)
        y = jnp.take_along_axis(x, col, axis=1)
        out_ref[...] = y[:, :SLICE]

    return pl.pallas_call(
        body,
        grid=grid,
        in_specs=[
            pl.BlockSpec((blk, d), lambda i: (i + off, 0)),
            pl.BlockSpec((blk, 1), lambda i: (i + off, 0)),
        ],
        out_specs=pl.BlockSpec((blk, SLICE), lambda i: (i, 0)),
        out_shape=jax.ShapeDtypeStruct((n_rows, SLICE), jnp.float32),
    )


def kernel(input_tensor, slices_index, slice_len):
    n, d = input_tensor.shape
    adj_idx = slices_index.astype(jnp.int32) + (
        jnp.asarray(slice_len, jnp.int32) - SLICE)

    n_sc = SC_ROWS
    num_workers = 32
    nc = 2
    sc_f = _sc_slice_gather(n_sc, d, n_sc // num_workers, 256, nc)
    sc_out = sc_f(input_tensor.reshape(n * d // L, L), adj_idx)

    tc_f = _tc_slice(n, d, n_sc, n - n_sc, TC_BLK)
    tc_out = tc_f(input_tensor, adj_idx.reshape(n, 1))

    return jnp.concatenate(
        [sc_out.reshape(n_sc, SLICE), tc_out], axis=0)


# speedup vs baseline: 1.8078x; 1.8078x over previous
import functools

import jax
import jax.numpy as jnp
from jax import lax
from jax.experimental import pallas as pl
from jax.experimental.pallas import tpu as pltpu
from jax.experimental.pallas import tpu_sc as plsc

SLICE = 64
L = 16


def _sc_slice_gather(n, d, rows_per_w, chunk_rows, nc):
    n_chunks = rows_per_w // chunk_rows
    assert n_chunks % 2 == 0
    groups = chunk_rows // L
    j_steps = SLICE // L

    mesh = plsc.VectorSubcoreMesh(core_axis_name="c", subcore_axis_name="s")

    @functools.partial(
        pl.kernel,
        mesh=mesh,
        compiler_params=pltpu.CompilerParams(needs_layout_passes=False),
        out_type=jax.ShapeDtypeStruct((n, SLICE), jnp.float32),
        scratch_types=[
            pltpu.VMEM((chunk_rows, d), jnp.float32),
            pltpu.VMEM((chunk_rows, d), jnp.float32),
            pltpu.VMEM((chunk_rows, SLICE), jnp.float32),
            pltpu.VMEM((chunk_rows, SLICE), jnp.float32),
            pltpu.VMEM((chunk_rows,), jnp.int32),
            pltpu.VMEM((chunk_rows,), jnp.int32),
            pltpu.SemaphoreType.DMA,
            pltpu.SemaphoreType.DMA,
            pltpu.SemaphoreType.DMA,
            pltpu.SemaphoreType.DMA,
        ],
    )
    def k(in_hbm, idx_hbm, out_hbm, in_v0, in_v1, out_v0, out_v1,
          idx_v0, idx_v1, sem_in0, sem_in1, sem_out0, sem_out1):
        in_v = (in_v0, in_v1)
        out_v = (out_v0, out_v1)
        idx_v = (idx_v0, idx_v1)
        sem_in = (sem_in0, sem_in1)
        sem_out = (sem_out0, sem_out1)
        wid = lax.axis_index("s") * nc + lax.axis_index("c")
        base_row = wid * rows_per_w
        iota = lax.iota(jnp.int32, L)

        def in_copy(c, b):
            row0 = base_row + c * chunk_rows
            return (
                pltpu.make_async_copy(
                    in_hbm.at[pl.ds(row0, chunk_rows)], in_v[b], sem_in[b]),
                pltpu.make_async_copy(
                    idx_hbm.at[pl.ds(row0, chunk_rows)], idx_v[b], sem_in[b]),
            )

        def out_copy(c, b):
            row0 = base_row + c * chunk_rows
            return pltpu.make_async_copy(
                out_v[b], out_hbm.at[pl.ds(row0, chunk_rows)], sem_out[b])

        def compute(b):
            @plsc.parallel_loop(0, groups, 1)
            def group_body(g):
                svec = idx_v[b][pl.ds(g * L, L)]
                for r in range(L):
                    row = g * L + r
                    s_b = jnp.take_along_axis(
                        svec, jnp.full((L,), r, jnp.int32), axis=0)
                    rvec = jnp.full((L,), row, jnp.int32)
                    col0 = s_b + iota
                    for j in range(j_steps):
                        vals = plsc.load_gather(
                            in_v[b], [rvec, col0 + (j * L)])
                        out_v[b][row, pl.ds(j * L, L)] = vals

        for b in range(2):
            for cp in in_copy(b, b):
                cp.start()

        def pair_body(i, carry):
            for b in range(2):
                c = i * 2 + b
                for cp in in_copy(c, b):
                    cp.wait()

                @pl.when(i > 0)
                def _():
                    out_copy(c, b).wait()

                compute(b)
                out_copy(c, b).start()

                @pl.when(c + 2 < n_chunks)
                def _():
                    for cp in in_copy(c + 2, b):
                        cp.start()
            return carry

        lax.fori_loop(0, n_chunks // 2, pair_body, 0)
        for b in range(2):
            out_copy(n_chunks - 2 + b, b).wait()

    return k


def kernel(input_tensor, slices_index, slice_len):
    n, d = input_tensor.shape
    adj_idx = slices_index.astype(jnp.int32) + (
        jnp.asarray(slice_len, jnp.int32) - SLICE)

    num_workers = 32
    nc = 2
    rows_per_w = n // num_workers
    chunk_rows = 128
    f = _sc_slice_gather(n, d, rows_per_w, chunk_rows, nc)
    return f(input_tensor, adj_idx)
